# baseline (speedup 1.0000x reference)
import jax, jax.numpy as jnp
from jax.experimental import pallas as pl

N = 8192
IN_DIM = 3
C = 4
H = 8
D = 64
NH = 4
BS = 128
NW = 8
MH = 256
OUT = 3
DC = D + C
DCP = 128


def _ln(x, g, b):
    m = x.mean(-1, keepdims=True)
    v = ((x - m) ** 2).mean(-1, keepdims=True)
    return (x - m) / jnp.sqrt(v + 1e-5) * g + b


def _attn_blocks(sq, sk, sv, qsq, ksq):
    G = sq.shape[0]

    def body(q_ref, k_ref, v_ref, qsq_ref, ksq_ref, so_ref, den_ref):
        q = q_ref[0]
        k = k_ref[0]
        v = v_ref[0]
        qsq = qsq_ref[0].T
        ksq = ksq_ref[0]
        cd = jnp.dot(q, k.T, preferred_element_type=jnp.float32)
        qk = jnp.exp(jnp.minimum(cd + qsq + ksq, 0.0))
        so_ref[0] = jnp.dot(qk, v, preferred_element_type=jnp.float32)
        den_ref[0] = jnp.sum(qk.T, axis=0, keepdims=True)

    return pl.pallas_call(
        body,
        grid=(G,),
        in_specs=[pl.BlockSpec((1, BS, DCP), lambda i: (i, 0, 0)),
                  pl.BlockSpec((1, BS, DCP), lambda i: (i, 0, 0)),
                  pl.BlockSpec((1, BS, D), lambda i: (i, 0, 0)),
                  pl.BlockSpec((1, 1, BS), lambda i: (i, 0, 0)),
                  pl.BlockSpec((1, 1, BS), lambda i: (i, 0, 0))],
        out_specs=[pl.BlockSpec((1, BS, D), lambda i: (i, 0, 0)),
                   pl.BlockSpec((1, 1, BS), lambda i: (i, 0, 0))],
        out_shape=[jax.ShapeDtypeStruct((G, BS, D), jnp.float32),
                   jax.ShapeDtypeStruct((G, 1, BS), jnp.float32)],
    )(sq, sk, sv, qsq, ksq)


def _attn_layer(x, coords, combined_shifts, p, i):
    n = x.shape[0]
    xn = _ln(x, p['ln%d_1_g' % i], p['ln%d_1_b' % i])
    q = (xn @ p['wq%d' % i]).reshape(n, H, D)
    k = (xn @ p['wk%d' % i]).reshape(n, H, D)
    v = (xn @ p['wv%d' % i]).reshape(n, H, D)
    w = p['w_rpe%d' % i].reshape(H, D, C - 1, NW)
    qw = jnp.exp(jnp.minimum(w.sum(axis=1), 50.0)).sum(axis=-1)
    new_qw = jnp.concatenate([qw[:, :1], qw], axis=-1)
    sqrt_w_r = jnp.sqrt(2.0 * new_qw)[None] * coords[:, None, :]
    q_hat = jnp.transpose(jnp.concatenate([q, sqrt_w_r], axis=-1), (1, 0, 2))
    k_hat = jnp.transpose(jnp.concatenate([k, sqrt_w_r], axis=-1), (1, 0, 2))
    value = jnp.transpose(v, (1, 0, 2))
    alpha = p['alpha%d' % i]
    q_hashed = jax.lax.stop_gradient(jnp.transpose(jnp.einsum('hnd,hdk->hnk', q_hat, alpha), (2, 0, 1)))
    k_hashed = jax.lax.stop_gradient(jnp.transpose(jnp.einsum('hnd,hdk->hnk', k_hat, alpha), (2, 0, 1)))
    max_shift = jnp.maximum(q_hashed.max(-1, keepdims=True), k_hashed.max(-1, keepdims=True))
    min_shift = jnp.minimum(q_hashed.min(-1, keepdims=True), k_hashed.min(-1, keepdims=True))
    hash_shift = max_shift - min_shift
    cs = combined_shifts * hash_shift
    q_hashed = q_hashed + cs
    k_hashed = k_hashed + cs
    q_pos = jnp.argsort(q_hashed, axis=-1)
    k_pos = jnp.argsort(k_hashed, axis=-1)

    def _gsel(arr, pos, d):
        ab = jnp.broadcast_to(arr[None], (NH, H, n, d))
        idx = jnp.broadcast_to(pos[..., None], (NH, H, n, d))
        return jnp.take_along_axis(ab, idx, axis=2)

    G = NH * H * (n // BS)
    q_hat_p = jnp.pad(q_hat, ((0, 0), (0, 0), (0, DCP - DC)))
    k_hat_p = jnp.pad(k_hat, ((0, 0), (0, 0), (0, DCP - DC)))
    sq = _gsel(q_hat_p, q_pos, DCP).reshape(G, BS, DCP)
    sk = _gsel(k_hat_p, k_pos, DCP).reshape(G, BS, DCP)
    sv = _gsel(value, k_pos, D).reshape(G, BS, D)
    qsq = (-0.5 * (sq[..., :DC] ** 2).sum(-1)).reshape(G, 1, BS)
    ksq = (-0.5 * (sk[..., :DC] ** 2).sum(-1)).reshape(G, 1, BS)

    so, den = _attn_blocks(sq, sk, sv, qsq, ksq)
    denom = den.reshape(NH, H, -1, BS, 1) + 1e-20

    iota = jnp.broadcast_to(jnp.arange(n, dtype=q_pos.dtype), q_pos.shape)
    q_rev = jnp.zeros_like(q_pos)
    q_rev = jnp.put_along_axis(q_rev, q_pos, iota, axis=-1, inplace=False)
    so_sq = so.reshape(NH, H, n, D)
    o = jnp.take_along_axis(so_sq, jnp.broadcast_to(q_rev[..., None], (NH, H, n, D)), axis=2)
    den_sq = denom.reshape(NH, H, n, 1)
    logits = jnp.take_along_axis(den_sq, q_rev[..., None], axis=2)
    aggr = (o.sum(0) / logits.sum(0)).reshape(-1, H * D)
    aggr = aggr @ p['outw%d' % i] + p['outb%d' % i]
    x = x + aggr
    xn2 = _ln(x, p['ln%d_2_g' % i], p['ln%d_2_b' % i])
    ff = jnp.maximum(xn2 @ p['ffw1_%d' % i] + p['ffb1_%d' % i], 0.0) @ p['ffw2_%d' % i] + p['ffb2_%d' % i]
    return x + ff


def kernel(x, combined_shifts, coords, unpad_seq, params):
    h = jnp.maximum(x @ params['fe_w1'] + params['fe_b1'], 0.0) @ params['fe_w2'] + params['fe_b2']
    enc = h
    h = _attn_layer(h, coords, combined_shifts, params, 1)
    enc = jnp.concatenate([enc, h], axis=-1)
    h = _attn_layer(h, coords, combined_shifts, params, 2)
    enc = jnp.concatenate([enc, h], axis=-1)
    z = enc @ params['W_w']
    m = z
    for j in range(4):
        m = jnp.tanh(_ln(m @ params['mo_w%d' % j] + params['mo_b%d' % j], params['mo_g%d' % j], params['mo_bb%d' % j]))
    m = m @ params['mo_w4'] + params['mo_b4']
    z = z + m
    return z @ params['op_w'] + params['op_b']


# speedup vs baseline: 1.2711x; 1.2711x over previous
import jax, jax.numpy as jnp
from jax.experimental import pallas as pl

N = 8192
IN_DIM = 3
C = 4
H = 8
D = 64
NH = 4
BS = 128
NW = 8
MH = 256
OUT = 3
DC = D + C
DCP = 128


def _ln(x, g, b):
    m = x.mean(-1, keepdims=True)
    v = ((x - m) ** 2).mean(-1, keepdims=True)
    return (x - m) / jnp.sqrt(v + 1e-5) * g + b


def _attn_blocks(sq, sk, sv, qsq, ksq):
    G = sq.shape[0]

    def body(q_ref, k_ref, v_ref, qsq_ref, ksq_ref, so_ref, den_ref):
        q = q_ref[0]
        k = k_ref[0]
        v = v_ref[0]
        qsq = qsq_ref[0].T
        ksq = ksq_ref[0]
        cd = jnp.dot(q, k.T, preferred_element_type=jnp.float32)
        qk = jnp.exp(jnp.minimum(cd + qsq + ksq, 0.0))
        so_ref[0] = jnp.dot(qk, v, preferred_element_type=jnp.float32)
        den_ref[0] = jnp.sum(qk.T, axis=0, keepdims=True)

    return pl.pallas_call(
        body,
        grid=(G,),
        in_specs=[pl.BlockSpec((1, BS, DCP), lambda i: (i, 0, 0)),
                  pl.BlockSpec((1, BS, DCP), lambda i: (i, 0, 0)),
                  pl.BlockSpec((1, BS, D), lambda i: (i, 0, 0)),
                  pl.BlockSpec((1, 1, BS), lambda i: (i, 0, 0)),
                  pl.BlockSpec((1, 1, BS), lambda i: (i, 0, 0))],
        out_specs=[pl.BlockSpec((1, BS, D), lambda i: (i, 0, 0)),
                   pl.BlockSpec((1, 1, BS), lambda i: (i, 0, 0))],
        out_shape=[jax.ShapeDtypeStruct((G, BS, D), jnp.float32),
                   jax.ShapeDtypeStruct((G, 1, BS), jnp.float32)],
    )(sq, sk, sv, qsq, ksq)


def _attn_layer(x, coords, combined_shifts, p, i):
    n = x.shape[0]
    xn = _ln(x, p['ln%d_1_g' % i], p['ln%d_1_b' % i])
    q = (xn @ p['wq%d' % i]).reshape(n, H, D)
    k = (xn @ p['wk%d' % i]).reshape(n, H, D)
    v = (xn @ p['wv%d' % i]).reshape(n, H, D)
    w = p['w_rpe%d' % i].reshape(H, D, C - 1, NW)
    qw = jnp.exp(jnp.minimum(w.sum(axis=1), 50.0)).sum(axis=-1)
    new_qw = jnp.concatenate([qw[:, :1], qw], axis=-1)
    sqrt_w_r = jnp.sqrt(2.0 * new_qw)[None] * coords[:, None, :]
    q_hat = jnp.transpose(jnp.concatenate([q, sqrt_w_r], axis=-1), (1, 0, 2))
    k_hat = jnp.transpose(jnp.concatenate([k, sqrt_w_r], axis=-1), (1, 0, 2))
    value = jnp.transpose(v, (1, 0, 2))
    alpha = p['alpha%d' % i]
    q_hashed = jax.lax.stop_gradient(jnp.transpose(jnp.einsum('hnd,hdk->hnk', q_hat, alpha), (2, 0, 1)))
    k_hashed = jax.lax.stop_gradient(jnp.transpose(jnp.einsum('hnd,hdk->hnk', k_hat, alpha), (2, 0, 1)))
    max_shift = jnp.maximum(q_hashed.max(-1, keepdims=True), k_hashed.max(-1, keepdims=True))
    min_shift = jnp.minimum(q_hashed.min(-1, keepdims=True), k_hashed.min(-1, keepdims=True))
    hash_shift = max_shift - min_shift
    cs = combined_shifts * hash_shift
    q_hashed = q_hashed + cs
    k_hashed = k_hashed + cs
    q_pos = jnp.argsort(q_hashed, axis=-1)
    k_pos = jnp.argsort(k_hashed, axis=-1)

    def _gsel(arr, pos, d):
        ab = jnp.broadcast_to(arr[None], (NH, H, n, d))
        idx = jnp.broadcast_to(pos[..., None], (NH, H, n, d))
        return jnp.take_along_axis(ab, idx, axis=2)

    s_query = _gsel(q_hat, q_pos, DC).reshape(NH, H, -1, BS, DC)
    s_key = _gsel(k_hat, k_pos, DC).reshape(NH, H, -1, BS, DC)
    s_value = _gsel(value, k_pos, D).reshape(NH, H, -1, BS, D)
    q_sq = -0.5 * (s_query ** 2).sum(-1, keepdims=True)
    k_sq = -0.5 * (s_key ** 2).sum(-1, keepdims=True)

    G = NH * H * (n // BS)
    sq = jnp.pad(s_query.reshape(G, BS, DC), ((0, 0), (0, 0), (0, DCP - DC)))
    sk = jnp.pad(s_key.reshape(G, BS, DC), ((0, 0), (0, 0), (0, DCP - DC)))
    sv = s_value.reshape(G, BS, D)
    qsq = q_sq.reshape(G, 1, BS)
    ksq = k_sq.reshape(G, 1, BS)

    so, den = _attn_blocks(sq, sk, sv, qsq, ksq)
    denom = den.reshape(NH, H, -1, BS, 1) + 1e-20

    q_rev = jnp.argsort(q_pos, axis=-1)
    so_sq = so.reshape(NH, H, n, D)
    o = jnp.take_along_axis(so_sq, jnp.broadcast_to(q_rev[..., None], (NH, H, n, D)), axis=2)
    den_sq = denom.reshape(NH, H, n, 1)
    logits = jnp.take_along_axis(den_sq, q_rev[..., None], axis=2)
    aggr = (o.sum(0) / logits.sum(0)).reshape(-1, H * D)
    aggr = aggr @ p['outw%d' % i] + p['outb%d' % i]
    x = x + aggr
    xn2 = _ln(x, p['ln%d_2_g' % i], p['ln%d_2_b' % i])
    ff = jnp.maximum(xn2 @ p['ffw1_%d' % i] + p['ffb1_%d' % i], 0.0) @ p['ffw2_%d' % i] + p['ffb2_%d' % i]
    return x + ff


def kernel(x, combined_shifts, coords, unpad_seq, params):
    h = jnp.maximum(x @ params['fe_w1'] + params['fe_b1'], 0.0) @ params['fe_w2'] + params['fe_b2']
    enc = h
    h = _attn_layer(h, coords, combined_shifts, params, 1)
    enc = jnp.concatenate([enc, h], axis=-1)
    h = _attn_layer(h, coords, combined_shifts, params, 2)
    enc = jnp.concatenate([enc, h], axis=-1)
    z = enc @ params['W_w']
    m = z
    for j in range(4):
        m = jnp.tanh(_ln(m @ params['mo_w%d' % j] + params['mo_b%d' % j], params['mo_g%d' % j], params['mo_bb%d' % j]))
    m = m @ params['mo_w4'] + params['mo_b4']
    z = z + m
    return z @ params['op_w'] + params['op_b']
